# baseline (speedup 1.0000x reference)
import functools

import jax
import jax.numpy as jnp
from jax import lax
from jax.experimental import pallas as pl
from jax.experimental.pallas import tpu as pltpu
from jax.experimental.pallas import tpu_sc as plsc

K_NN = 16
B = 4
N = 2048
TC_SHARE = 1152
ROWS = 384

L = 16
NW = 32
W_PER_BATCH = NW // B
SC_SHARE = N - TC_SHARE
ROWS_PER_W = SC_SHARE // W_PER_BATCH
NV = N // L
CAP = 256


def _tc_block(fx_nat_ref, fx_t_ref, kp_nat_ref, kp_t_ref, out_ref):
    i = pl.program_id(1)
    r0 = i * ROWS

    fxb = fx_t_ref[0]
    kpb = kp_t_ref[0]
    fx_rows = fx_nat_ref[0, pl.ds(r0, ROWS), :]
    kp_rows = kp_nat_ref[0, pl.ds(r0, ROWS), :]

    def sq_dist(rows, pts_t):
        a2 = jnp.sum(rows * rows, axis=1, keepdims=True)
        b2 = jnp.sum(pts_t * pts_t, axis=0, keepdims=True)
        ab = jax.lax.dot_general(
            rows, pts_t, (((1,), (0,)), ((), ())),
            preferred_element_type=jnp.float32,
            precision=jax.lax.Precision.HIGHEST)
        return jnp.maximum(a2 + b2 - 2.0 * ab, 0.0)

    d_fixed = sq_dist(fx_rows, fxb)
    d_kpt = sq_dist(kp_rows, kpb)

    m = jnp.min(d_fixed, axis=1, keepdims=True)
    for _ in range(K_NN - 1):
        m = jnp.min(jnp.where(d_fixed > m, d_fixed, jnp.inf),
                    axis=1, keepdims=True)

    diff = d_fixed - d_kpt
    s = diff * diff
    le = d_fixed <= m
    eqm = d_fixed == m
    s_le = jnp.sum(jnp.where(le, s, 0.0), axis=1, keepdims=True)
    s_eq = jnp.sum(jnp.where(eqm, s, 0.0), axis=1, keepdims=True)
    cnt_le = jnp.sum(jnp.where(le, 1.0, 0.0), axis=1, keepdims=True)
    cnt_eq = jnp.sum(jnp.where(eqm, 1.0, 0.0), axis=1, keepdims=True)
    cnt_lt = cnt_le - cnt_eq
    w = jnp.clip((K_NN - cnt_lt) / jnp.maximum(cnt_eq, 1.0), 0.0, 1.0)
    loss = jnp.sum(s_le - (1.0 - w) * s_eq)
    out_ref[...] = loss.reshape(1, 1, 1, 1)


def _sc_body(fxx_h, fxy_h, fxz_h, kpx_h, kpy_h, kpz_h, out_h,
             fxx, fxy, fxz, kpx, kpy, kpz, dbuf, candv, candi, acc):
    wid = lax.axis_index("s") * 2 + lax.axis_index("c")
    b = wid // W_PER_BATCH
    row0 = TC_SHARE + (wid % W_PER_BATCH) * ROWS_PER_W

    pltpu.sync_copy(fxx_h.at[b], fxx)
    pltpu.sync_copy(fxy_h.at[b], fxy)
    pltpu.sync_copy(fxz_h.at[b], fxz)
    pltpu.sync_copy(kpx_h.at[b], kpx)
    pltpu.sync_copy(kpy_h.at[b], kpy)
    pltpu.sync_copy(kpz_h.at[b], kpz)

    lane = lax.iota(jnp.int32, L)
    inf16 = jnp.full((L,), jnp.inf, jnp.float32)
    zero16 = jnp.zeros((L,), jnp.float32)
    for j in range(CAP // L):
        candi[pl.ds(j * L, L)] = lax.iota(jnp.int32, L)

    acc[...] = zero16

    def row_body(r, acc_v):
        ridx = jnp.full((L,), row0 + r, jnp.int32)
        xi = plsc.load_gather(fxx, [ridx])
        yi = plsc.load_gather(fxy, [ridx])
        zi = plsc.load_gather(fxz, [ridx])

        @plsc.parallel_loop(0, N, step=L, unroll=16, carry=inf16)
        def g(v, gc):
            dx = fxx[pl.ds(v, L)] - xi
            dy = fxy[pl.ds(v, L)] - yi
            dz = fxz[pl.ds(v, L)] - zi
            d = dx * dx + dy * dy + dz * dz
            dbuf[pl.ds(v, L)] = d
            return jnp.minimum(gc, d)
        t = jnp.full((L,), jnp.max(g), jnp.float32)

        for j in range(CAP // L):
            candv[pl.ds(j * L, L)] = inf16

        @plsc.parallel_loop(0, N, step=L, unroll=8,
                            carry=jnp.zeros((L,), jnp.int32))
        def off(v, offc):
            d = dbuf[pl.ds(v, L)]
            m = d <= t
            pos = offc + plsc.cumsum(jnp.ones((L,), jnp.int32), mask=m) - 1
            m = jnp.logical_and(m, pos < CAP)
            plsc.store_scatter(candv, [pos], d, mask=m)
            plsc.store_scatter(candi, [pos], v + lane, mask=m)
            return offc + plsc.all_reduce_population_count(m)
        cnt = jnp.max(off)
        nv = jnp.minimum((cnt + L - 1) // L, CAP // L)

        bk, bv = plsc.sort_key_val(candv[pl.ds(0, L)], candi[pl.ds(0, L)])

        def merge_body(j, carry):
            bk, bv = carry
            ck, cv = plsc.sort_key_val(candv[pl.ds(j * L, L)],
                                       candi[pl.ds(j * L, L)])
            ck = lax.rev(ck, (0,))
            cv = lax.rev(cv, (0,))
            m = bk <= ck
            lo_k = jnp.where(m, bk, ck)
            lo_v = jnp.where(m, bv, cv)
            r = plsc.sort_key_val(lo_k, lo_v)
            return (r[0], r[1])

        bk, bv = lax.fori_loop(1, nv, merge_body, (bk, bv))

        kxi = plsc.load_gather(kpx, [ridx])
        kyi = plsc.load_gather(kpy, [ridx])
        kzi = plsc.load_gather(kpz, [ridx])
        gx = plsc.load_gather(kpx, [bv])
        gy = plsc.load_gather(kpy, [bv])
        gz = plsc.load_gather(kpz, [bv])
        ex = gx - kxi
        ey = gy - kyi
        ez = gz - kzi
        dk = ex * ex + ey * ey + ez * ez
        diff = bk - dk
        return acc_v + diff * diff

    acc_v = lax.fori_loop(0, ROWS_PER_W, row_body, zero16)
    acc[...] = acc_v
    pltpu.sync_copy(acc, out_h.at[wid])


def kernel(keypt, fixed_frame):
    fx_t = jnp.swapaxes(fixed_frame, 1, 2)
    kp_t = jnp.swapaxes(keypt, 1, 2)

    tc_partial = pl.pallas_call(
        _tc_block,
        grid=(B, TC_SHARE // ROWS),
        in_specs=[
            pl.BlockSpec((1, N, 3), lambda b, i: (b, 0, 0)),
            pl.BlockSpec((1, 3, N), lambda b, i: (b, 0, 0)),
            pl.BlockSpec((1, N, 3), lambda b, i: (b, 0, 0)),
            pl.BlockSpec((1, 3, N), lambda b, i: (b, 0, 0)),
        ],
        out_specs=pl.BlockSpec((1, 1, 1, 1), lambda b, i: (b, i, 0, 0)),
        out_shape=jax.ShapeDtypeStruct((B, TC_SHARE // ROWS, 1, 1),
                                       jnp.float32),
    )(fixed_frame, fx_t, keypt, kp_t)

    mesh = plsc.VectorSubcoreMesh(core_axis_name="c", subcore_axis_name="s")
    sc_partial = pl.kernel(
        _sc_body,
        mesh=mesh,
        out_type=jax.ShapeDtypeStruct((NW, L), jnp.float32),
        scratch_types=[
            pltpu.VMEM((N,), jnp.float32),
            pltpu.VMEM((N,), jnp.float32),
            pltpu.VMEM((N,), jnp.float32),
            pltpu.VMEM((N,), jnp.float32),
            pltpu.VMEM((N,), jnp.float32),
            pltpu.VMEM((N,), jnp.float32),
            pltpu.VMEM((N,), jnp.float32),
            pltpu.VMEM((CAP,), jnp.float32),
            pltpu.VMEM((CAP,), jnp.int32),
            pltpu.VMEM((L,), jnp.float32),
        ],
        compiler_params=pltpu.CompilerParams(needs_layout_passes=False),
    )(fixed_frame[:, :, 0], fixed_frame[:, :, 1], fixed_frame[:, :, 2],
      keypt[:, :, 0], keypt[:, :, 1], keypt[:, :, 2])

    return (jnp.sum(tc_partial) + jnp.sum(sc_partial)) / (B * N)


# speedup vs baseline: 1.0318x; 1.0318x over previous
import functools

import jax
import jax.numpy as jnp
from jax import lax
from jax.experimental import pallas as pl
from jax.experimental.pallas import tpu as pltpu
from jax.experimental.pallas import tpu_sc as plsc

K_NN = 16
B = 4
N = 2048
TC_SHARE = 1088
ROWS = 272

L = 16
NW = 32
W_PER_BATCH = NW // B
SC_SHARE = N - TC_SHARE
ROWS_PER_W = SC_SHARE // W_PER_BATCH
NV = N // L
CAP = 256


def _tc_block(fx_nat_ref, fx_t_ref, kp_nat_ref, kp_t_ref, out_ref):
    i = pl.program_id(1)
    r0 = i * ROWS

    fxb = fx_t_ref[0]
    kpb = kp_t_ref[0]
    fx_rows = fx_nat_ref[0, pl.ds(r0, ROWS), :]
    kp_rows = kp_nat_ref[0, pl.ds(r0, ROWS), :]

    def sq_dist(rows, pts_t):
        a2 = jnp.sum(rows * rows, axis=1, keepdims=True)
        b2 = jnp.sum(pts_t * pts_t, axis=0, keepdims=True)
        ab = jax.lax.dot_general(
            rows, pts_t, (((1,), (0,)), ((), ())),
            preferred_element_type=jnp.float32,
            precision=jax.lax.Precision.HIGHEST)
        return jnp.maximum(a2 + b2 - 2.0 * ab, 0.0)

    d_fixed = sq_dist(fx_rows, fxb)
    d_kpt = sq_dist(kp_rows, kpb)

    m = jnp.min(d_fixed, axis=1, keepdims=True)
    for _ in range(K_NN - 1):
        m = jnp.min(jnp.where(d_fixed > m, d_fixed, jnp.inf),
                    axis=1, keepdims=True)

    diff = d_fixed - d_kpt
    s = diff * diff
    le = d_fixed <= m
    eqm = d_fixed == m
    s_le = jnp.sum(jnp.where(le, s, 0.0), axis=1, keepdims=True)
    s_eq = jnp.sum(jnp.where(eqm, s, 0.0), axis=1, keepdims=True)
    cnt_le = jnp.sum(jnp.where(le, 1.0, 0.0), axis=1, keepdims=True)
    cnt_eq = jnp.sum(jnp.where(eqm, 1.0, 0.0), axis=1, keepdims=True)
    cnt_lt = cnt_le - cnt_eq
    w = jnp.clip((K_NN - cnt_lt) / jnp.maximum(cnt_eq, 1.0), 0.0, 1.0)
    loss = jnp.sum(s_le - (1.0 - w) * s_eq)
    out_ref[...] = loss.reshape(1, 1, 1, 1)


def _sc_body(fxx_h, fxy_h, fxz_h, kpx_h, kpy_h, kpz_h, out_h,
             fxx, fxy, fxz, kpx, kpy, kpz, dbuf, candv, candi, acc):
    wid = lax.axis_index("s") * 2 + lax.axis_index("c")
    b = wid // W_PER_BATCH
    row0 = TC_SHARE + (wid % W_PER_BATCH) * ROWS_PER_W

    pltpu.sync_copy(fxx_h.at[b], fxx)
    pltpu.sync_copy(fxy_h.at[b], fxy)
    pltpu.sync_copy(fxz_h.at[b], fxz)
    pltpu.sync_copy(kpx_h.at[b], kpx)
    pltpu.sync_copy(kpy_h.at[b], kpy)
    pltpu.sync_copy(kpz_h.at[b], kpz)

    lane = lax.iota(jnp.int32, L)
    inf16 = jnp.full((L,), jnp.inf, jnp.float32)
    zero16 = jnp.zeros((L,), jnp.float32)
    for j in range(CAP // L):
        candi[pl.ds(j * L, L)] = lax.iota(jnp.int32, L)

    acc[...] = zero16

    def row_body(r, acc_v):
        ridx = jnp.full((L,), row0 + r, jnp.int32)
        xi = plsc.load_gather(fxx, [ridx])
        yi = plsc.load_gather(fxy, [ridx])
        zi = plsc.load_gather(fxz, [ridx])

        @plsc.parallel_loop(0, N, step=L, unroll=16, carry=inf16)
        def g(v, gc):
            dx = fxx[pl.ds(v, L)] - xi
            dy = fxy[pl.ds(v, L)] - yi
            dz = fxz[pl.ds(v, L)] - zi
            d = dx * dx + dy * dy + dz * dz
            dbuf[pl.ds(v, L)] = d
            return jnp.minimum(gc, d)
        t = jnp.full((L,), jnp.max(g), jnp.float32)

        for j in range(CAP // L):
            candv[pl.ds(j * L, L)] = inf16

        @plsc.parallel_loop(0, N, step=L, unroll=8,
                            carry=jnp.zeros((L,), jnp.int32))
        def off(v, offc):
            d = dbuf[pl.ds(v, L)]
            m = d <= t
            pos = offc + plsc.cumsum(jnp.ones((L,), jnp.int32), mask=m) - 1
            m = jnp.logical_and(m, pos < CAP)
            plsc.store_scatter(candv, [pos], d, mask=m)
            plsc.store_scatter(candi, [pos], v + lane, mask=m)
            return offc + plsc.all_reduce_population_count(m)
        cnt = jnp.max(off)
        nv = jnp.minimum((cnt + L - 1) // L, CAP // L)

        bk, bv = plsc.sort_key_val(candv[pl.ds(0, L)], candi[pl.ds(0, L)])

        def merge_body(j, carry):
            bk, bv = carry
            ck, cv = plsc.sort_key_val(candv[pl.ds(j * L, L)],
                                       candi[pl.ds(j * L, L)])
            ck = lax.rev(ck, (0,))
            cv = lax.rev(cv, (0,))
            m = bk <= ck
            lo_k = jnp.where(m, bk, ck)
            lo_v = jnp.where(m, bv, cv)
            r = plsc.sort_key_val(lo_k, lo_v)
            return (r[0], r[1])

        bk, bv = lax.fori_loop(1, nv, merge_body, (bk, bv))

        kxi = plsc.load_gather(kpx, [ridx])
        kyi = plsc.load_gather(kpy, [ridx])
        kzi = plsc.load_gather(kpz, [ridx])
        gx = plsc.load_gather(kpx, [bv])
        gy = plsc.load_gather(kpy, [bv])
        gz = plsc.load_gather(kpz, [bv])
        ex = gx - kxi
        ey = gy - kyi
        ez = gz - kzi
        dk = ex * ex + ey * ey + ez * ez
        diff = bk - dk
        return acc_v + diff * diff

    acc_v = lax.fori_loop(0, ROWS_PER_W, row_body, zero16)
    acc[...] = acc_v
    pltpu.sync_copy(acc, out_h.at[wid])


def kernel(keypt, fixed_frame):
    fx_t = jnp.swapaxes(fixed_frame, 1, 2)
    kp_t = jnp.swapaxes(keypt, 1, 2)

    tc_partial = pl.pallas_call(
        _tc_block,
        grid=(B, TC_SHARE // ROWS),
        in_specs=[
            pl.BlockSpec((1, N, 3), lambda b, i: (b, 0, 0)),
            pl.BlockSpec((1, 3, N), lambda b, i: (b, 0, 0)),
            pl.BlockSpec((1, N, 3), lambda b, i: (b, 0, 0)),
            pl.BlockSpec((1, 3, N), lambda b, i: (b, 0, 0)),
        ],
        out_specs=pl.BlockSpec((1, 1, 1, 1), lambda b, i: (b, i, 0, 0)),
        out_shape=jax.ShapeDtypeStruct((B, TC_SHARE // ROWS, 1, 1),
                                       jnp.float32),
    )(fixed_frame, fx_t, keypt, kp_t)

    mesh = plsc.VectorSubcoreMesh(core_axis_name="c", subcore_axis_name="s")
    sc_partial = pl.kernel(
        _sc_body,
        mesh=mesh,
        out_type=jax.ShapeDtypeStruct((NW, L), jnp.float32),
        scratch_types=[
            pltpu.VMEM((N,), jnp.float32),
            pltpu.VMEM((N,), jnp.float32),
            pltpu.VMEM((N,), jnp.float32),
            pltpu.VMEM((N,), jnp.float32),
            pltpu.VMEM((N,), jnp.float32),
            pltpu.VMEM((N,), jnp.float32),
            pltpu.VMEM((N,), jnp.float32),
            pltpu.VMEM((CAP,), jnp.float32),
            pltpu.VMEM((CAP,), jnp.int32),
            pltpu.VMEM((L,), jnp.float32),
        ],
        compiler_params=pltpu.CompilerParams(needs_layout_passes=False),
    )(fixed_frame[:, :, 0], fixed_frame[:, :, 1], fixed_frame[:, :, 2],
      keypt[:, :, 0], keypt[:, :, 1], keypt[:, :, 2])

    return (jnp.sum(tc_partial) + jnp.sum(sc_partial)) / (B * N)


# speedup vs baseline: 1.0872x; 1.0537x over previous
import functools

import jax
import jax.numpy as jnp
from jax import lax
from jax.experimental import pallas as pl
from jax.experimental.pallas import tpu as pltpu
from jax.experimental.pallas import tpu_sc as plsc

K_NN = 16
B = 4
N = 2048
TC_SHARE = 1088
ROWS = 272

L = 16
NW = 32
W_PER_BATCH = NW // B
SC_SHARE = N - TC_SHARE
ROWS_PER_W = SC_SHARE // W_PER_BATCH
NV = N // L
CAP = 256


def _tc_block(fx_nat_ref, fx_t_ref, kp_nat_ref, kp_t_ref, out_ref):
    i = pl.program_id(1)
    r0 = i * ROWS

    fxb = fx_t_ref[0]
    kpb = kp_t_ref[0]
    fx_rows = fx_nat_ref[0, pl.ds(r0, ROWS), :]
    kp_rows = kp_nat_ref[0, pl.ds(r0, ROWS), :]

    def sq_dist(rows, pts_t):
        a2 = jnp.sum(rows * rows, axis=1, keepdims=True)
        b2 = jnp.sum(pts_t * pts_t, axis=0, keepdims=True)
        ab = jax.lax.dot_general(
            rows, pts_t, (((1,), (0,)), ((), ())),
            preferred_element_type=jnp.float32)
        return jnp.maximum(a2 + b2 - 2.0 * ab, 0.0)

    d_fixed = sq_dist(fx_rows, fxb)
    d_kpt = sq_dist(kp_rows, kpb)

    m = jnp.min(d_fixed, axis=1, keepdims=True)
    for _ in range(K_NN - 1):
        m = jnp.min(jnp.where(d_fixed > m, d_fixed, jnp.inf),
                    axis=1, keepdims=True)

    diff = d_fixed - d_kpt
    s = diff * diff
    le = d_fixed <= m
    eqm = d_fixed == m
    s_le = jnp.sum(jnp.where(le, s, 0.0), axis=1, keepdims=True)
    s_eq = jnp.sum(jnp.where(eqm, s, 0.0), axis=1, keepdims=True)
    cnt_le = jnp.sum(jnp.where(le, 1.0, 0.0), axis=1, keepdims=True)
    cnt_eq = jnp.sum(jnp.where(eqm, 1.0, 0.0), axis=1, keepdims=True)
    cnt_lt = cnt_le - cnt_eq
    w = jnp.clip((K_NN - cnt_lt) / jnp.maximum(cnt_eq, 1.0), 0.0, 1.0)
    loss = jnp.sum(s_le - (1.0 - w) * s_eq)
    out_ref[...] = loss.reshape(1, 1, 1, 1)


def _sc_body(fxx_h, fxy_h, fxz_h, kpx_h, kpy_h, kpz_h, out_h,
             fxx, fxy, fxz, kpx, kpy, kpz, dbuf, candv, candi, acc):
    wid = lax.axis_index("s") * 2 + lax.axis_index("c")
    b = wid // W_PER_BATCH
    row0 = TC_SHARE + (wid % W_PER_BATCH) * ROWS_PER_W

    pltpu.sync_copy(fxx_h.at[b], fxx)
    pltpu.sync_copy(fxy_h.at[b], fxy)
    pltpu.sync_copy(fxz_h.at[b], fxz)
    pltpu.sync_copy(kpx_h.at[b], kpx)
    pltpu.sync_copy(kpy_h.at[b], kpy)
    pltpu.sync_copy(kpz_h.at[b], kpz)

    lane = lax.iota(jnp.int32, L)
    inf16 = jnp.full((L,), jnp.inf, jnp.float32)
    zero16 = jnp.zeros((L,), jnp.float32)
    for j in range(CAP // L):
        candi[pl.ds(j * L, L)] = lax.iota(jnp.int32, L)

    acc[...] = zero16

    def row_body(r, acc_v):
        ridx = jnp.full((L,), row0 + r, jnp.int32)
        xi = plsc.load_gather(fxx, [ridx])
        yi = plsc.load_gather(fxy, [ridx])
        zi = plsc.load_gather(fxz, [ridx])

        @plsc.parallel_loop(0, N, step=L, unroll=16, carry=inf16)
        def g(v, gc):
            dx = fxx[pl.ds(v, L)] - xi
            dy = fxy[pl.ds(v, L)] - yi
            dz = fxz[pl.ds(v, L)] - zi
            d = dx * dx + dy * dy + dz * dz
            dbuf[pl.ds(v, L)] = d
            return jnp.minimum(gc, d)
        t = jnp.full((L,), jnp.max(g), jnp.float32)

        for j in range(CAP // L):
            candv[pl.ds(j * L, L)] = inf16

        @plsc.parallel_loop(0, N, step=L, unroll=8,
                            carry=jnp.zeros((L,), jnp.int32))
        def off(v, offc):
            d = dbuf[pl.ds(v, L)]
            m = d <= t
            pos = offc + plsc.cumsum(jnp.ones((L,), jnp.int32), mask=m) - 1
            m = jnp.logical_and(m, pos < CAP)
            plsc.store_scatter(candv, [pos], d, mask=m)
            plsc.store_scatter(candi, [pos], v + lane, mask=m)
            return offc + plsc.all_reduce_population_count(m)
        cnt = jnp.max(off)
        nv = jnp.minimum((cnt + L - 1) // L, CAP // L)

        bk, bv = plsc.sort_key_val(candv[pl.ds(0, L)], candi[pl.ds(0, L)])

        def merge_body(j, carry):
            bk, bv = carry
            ck, cv = plsc.sort_key_val(candv[pl.ds(j * L, L)],
                                       candi[pl.ds(j * L, L)])
            ck = lax.rev(ck, (0,))
            cv = lax.rev(cv, (0,))
            m = bk <= ck
            lo_k = jnp.where(m, bk, ck)
            lo_v = jnp.where(m, bv, cv)
            r = plsc.sort_key_val(lo_k, lo_v)
            return (r[0], r[1])

        bk, bv = lax.fori_loop(1, nv, merge_body, (bk, bv))

        kxi = plsc.load_gather(kpx, [ridx])
        kyi = plsc.load_gather(kpy, [ridx])
        kzi = plsc.load_gather(kpz, [ridx])
        gx = plsc.load_gather(kpx, [bv])
        gy = plsc.load_gather(kpy, [bv])
        gz = plsc.load_gather(kpz, [bv])
        ex = gx - kxi
        ey = gy - kyi
        ez = gz - kzi
        dk = ex * ex + ey * ey + ez * ez
        diff = bk - dk
        return acc_v + diff * diff

    acc_v = lax.fori_loop(0, ROWS_PER_W, row_body, zero16)
    acc[...] = acc_v
    pltpu.sync_copy(acc, out_h.at[wid])


def kernel(keypt, fixed_frame):
    fx_t = jnp.swapaxes(fixed_frame, 1, 2)
    kp_t = jnp.swapaxes(keypt, 1, 2)

    tc_partial = pl.pallas_call(
        _tc_block,
        grid=(B, TC_SHARE // ROWS),
        in_specs=[
            pl.BlockSpec((1, N, 3), lambda b, i: (b, 0, 0)),
            pl.BlockSpec((1, 3, N), lambda b, i: (b, 0, 0)),
            pl.BlockSpec((1, N, 3), lambda b, i: (b, 0, 0)),
            pl.BlockSpec((1, 3, N), lambda b, i: (b, 0, 0)),
        ],
        out_specs=pl.BlockSpec((1, 1, 1, 1), lambda b, i: (b, i, 0, 0)),
        out_shape=jax.ShapeDtypeStruct((B, TC_SHARE // ROWS, 1, 1),
                                       jnp.float32),
    )(fixed_frame, fx_t, keypt, kp_t)

    mesh = plsc.VectorSubcoreMesh(core_axis_name="c", subcore_axis_name="s")
    sc_partial = pl.kernel(
        _sc_body,
        mesh=mesh,
        out_type=jax.ShapeDtypeStruct((NW, L), jnp.float32),
        scratch_types=[
            pltpu.VMEM((N,), jnp.float32),
            pltpu.VMEM((N,), jnp.float32),
            pltpu.VMEM((N,), jnp.float32),
            pltpu.VMEM((N,), jnp.float32),
            pltpu.VMEM((N,), jnp.float32),
            pltpu.VMEM((N,), jnp.float32),
            pltpu.VMEM((N,), jnp.float32),
            pltpu.VMEM((CAP,), jnp.float32),
            pltpu.VMEM((CAP,), jnp.int32),
            pltpu.VMEM((L,), jnp.float32),
        ],
        compiler_params=pltpu.CompilerParams(needs_layout_passes=False),
    )(fixed_frame[:, :, 0], fixed_frame[:, :, 1], fixed_frame[:, :, 2],
      keypt[:, :, 0], keypt[:, :, 1], keypt[:, :, 2])

    return (jnp.sum(tc_partial) + jnp.sum(sc_partial)) / (B * N)


# speedup vs baseline: 1.0989x; 1.0107x over previous
import functools

import jax
import jax.numpy as jnp
from jax import lax
from jax.experimental import pallas as pl
from jax.experimental.pallas import tpu as pltpu
from jax.experimental.pallas import tpu_sc as plsc

K_NN = 16
B = 4
N = 2048
TC_SHARE = 1088
ROWS = 272

L = 16
NW = 32
W_PER_BATCH = NW // B
SC_SHARE = N - TC_SHARE
ROWS_PER_W = SC_SHARE // W_PER_BATCH
NV = N // L
CAP = 256


def _tc_block(fx_nat_ref, fx_t_ref, kp_nat_ref, kp_t_ref, out_ref):
    i = pl.program_id(1)
    r0 = i * ROWS

    fxb = fx_t_ref[0]
    kpb = kp_t_ref[0]
    fx_rows = fx_nat_ref[0, pl.ds(r0, ROWS), :]
    kp_rows = kp_nat_ref[0, pl.ds(r0, ROWS), :]

    def sq_dist(rows, pts_t):
        a2 = jnp.sum(rows * rows, axis=1, keepdims=True)
        b2 = jnp.sum(pts_t * pts_t, axis=0, keepdims=True)
        ab = jax.lax.dot_general(
            rows, pts_t, (((1,), (0,)), ((), ())),
            preferred_element_type=jnp.float32)
        return jnp.maximum(a2 + b2 - 2.0 * ab, 0.0)

    d_fixed = sq_dist(fx_rows, fxb)
    d_kpt = sq_dist(kp_rows, kpb)

    m = jnp.min(d_fixed, axis=1, keepdims=True)
    for _ in range(K_NN - 1):
        m = jnp.min(jnp.where(d_fixed > m, d_fixed, jnp.inf),
                    axis=1, keepdims=True)

    diff = d_fixed - d_kpt
    s = diff * diff
    le = d_fixed <= m
    eqm = d_fixed == m
    s_le = jnp.sum(jnp.where(le, s, 0.0), axis=1, keepdims=True)
    s_eq = jnp.sum(jnp.where(eqm, s, 0.0), axis=1, keepdims=True)
    cnt_le = jnp.sum(jnp.where(le, 1.0, 0.0), axis=1, keepdims=True)
    cnt_eq = jnp.sum(jnp.where(eqm, 1.0, 0.0), axis=1, keepdims=True)
    cnt_lt = cnt_le - cnt_eq
    w = jnp.clip((K_NN - cnt_lt) / jnp.maximum(cnt_eq, 1.0), 0.0, 1.0)
    loss = jnp.sum(s_le - (1.0 - w) * s_eq)
    out_ref[...] = loss.reshape(1, 1, 1, 1)


def _sc_body(fx_t_h, kp_t_h, out_h,
             fxv, kpv, dbuf, candv, candi, acc):
    wid = lax.axis_index("s") * 2 + lax.axis_index("c")
    b = wid // W_PER_BATCH
    row0 = TC_SHARE + (wid % W_PER_BATCH) * ROWS_PER_W

    pltpu.sync_copy(fx_t_h.at[b], fxv)
    pltpu.sync_copy(kp_t_h.at[b], kpv)
    zero_i = jnp.zeros((L,), jnp.int32)
    one_i = jnp.full((L,), 1, jnp.int32)
    two_i = jnp.full((L,), 2, jnp.int32)

    lane = lax.iota(jnp.int32, L)
    inf16 = jnp.full((L,), jnp.inf, jnp.float32)
    zero16 = jnp.zeros((L,), jnp.float32)
    for j in range(CAP // L):
        candi[pl.ds(j * L, L)] = lax.iota(jnp.int32, L)

    acc[...] = zero16

    def row_body(r, acc_v):
        ridx = jnp.full((L,), row0 + r, jnp.int32)
        xi = plsc.load_gather(fxv, [zero_i, ridx])
        yi = plsc.load_gather(fxv, [one_i, ridx])
        zi = plsc.load_gather(fxv, [two_i, ridx])

        @plsc.parallel_loop(0, N, step=L, unroll=16, carry=inf16)
        def g(v, gc):
            dx = fxv[0, pl.ds(v, L)] - xi
            dy = fxv[1, pl.ds(v, L)] - yi
            dz = fxv[2, pl.ds(v, L)] - zi
            d = dx * dx + dy * dy + dz * dz
            dbuf[pl.ds(v, L)] = d
            return jnp.minimum(gc, d)
        t = jnp.full((L,), jnp.max(g), jnp.float32)

        for j in range(CAP // L):
            candv[pl.ds(j * L, L)] = inf16

        @plsc.parallel_loop(0, N, step=L, unroll=8,
                            carry=jnp.zeros((L,), jnp.int32))
        def off(v, offc):
            d = dbuf[pl.ds(v, L)]
            m = d <= t
            pos = offc + plsc.cumsum(jnp.ones((L,), jnp.int32), mask=m) - 1
            m = jnp.logical_and(m, pos < CAP)
            plsc.store_scatter(candv, [pos], d, mask=m)
            plsc.store_scatter(candi, [pos], v + lane, mask=m)
            return offc + plsc.all_reduce_population_count(m)
        cnt = jnp.max(off)
        nv = jnp.minimum((cnt + L - 1) // L, CAP // L)

        bk, bv = plsc.sort_key_val(candv[pl.ds(0, L)], candi[pl.ds(0, L)])

        def merge_body(j, carry):
            bk, bv = carry
            ck, cv = plsc.sort_key_val(candv[pl.ds(j * L, L)],
                                       candi[pl.ds(j * L, L)])
            ck = lax.rev(ck, (0,))
            cv = lax.rev(cv, (0,))
            m = bk <= ck
            lo_k = jnp.where(m, bk, ck)
            lo_v = jnp.where(m, bv, cv)
            r = plsc.sort_key_val(lo_k, lo_v)
            return (r[0], r[1])

        bk, bv = lax.fori_loop(1, nv, merge_body, (bk, bv))

        kxi = plsc.load_gather(kpv, [zero_i, ridx])
        kyi = plsc.load_gather(kpv, [one_i, ridx])
        kzi = plsc.load_gather(kpv, [two_i, ridx])
        gx = plsc.load_gather(kpv, [zero_i, bv])
        gy = plsc.load_gather(kpv, [one_i, bv])
        gz = plsc.load_gather(kpv, [two_i, bv])
        ex = gx - kxi
        ey = gy - kyi
        ez = gz - kzi
        dk = ex * ex + ey * ey + ez * ez
        diff = bk - dk
        return acc_v + diff * diff

    acc_v = lax.fori_loop(0, ROWS_PER_W, row_body, zero16)
    acc[...] = acc_v
    pltpu.sync_copy(acc, out_h.at[wid])


def kernel(keypt, fixed_frame):
    fx_t = jnp.swapaxes(fixed_frame, 1, 2)
    kp_t = jnp.swapaxes(keypt, 1, 2)

    tc_partial = pl.pallas_call(
        _tc_block,
        grid=(B, TC_SHARE // ROWS),
        in_specs=[
            pl.BlockSpec((1, N, 3), lambda b, i: (b, 0, 0)),
            pl.BlockSpec((1, 3, N), lambda b, i: (b, 0, 0)),
            pl.BlockSpec((1, N, 3), lambda b, i: (b, 0, 0)),
            pl.BlockSpec((1, 3, N), lambda b, i: (b, 0, 0)),
        ],
        out_specs=pl.BlockSpec((1, 1, 1, 1), lambda b, i: (b, i, 0, 0)),
        out_shape=jax.ShapeDtypeStruct((B, TC_SHARE // ROWS, 1, 1),
                                       jnp.float32),
    )(fixed_frame, fx_t, keypt, kp_t)

    mesh = plsc.VectorSubcoreMesh(core_axis_name="c", subcore_axis_name="s")
    sc_partial = pl.kernel(
        _sc_body,
        mesh=mesh,
        out_type=jax.ShapeDtypeStruct((NW, L), jnp.float32),
        scratch_types=[
            pltpu.VMEM((3, N), jnp.float32),
            pltpu.VMEM((3, N), jnp.float32),
            pltpu.VMEM((N,), jnp.float32),
            pltpu.VMEM((CAP,), jnp.float32),
            pltpu.VMEM((CAP,), jnp.int32),
            pltpu.VMEM((L,), jnp.float32),
        ],
        compiler_params=pltpu.CompilerParams(needs_layout_passes=False),
    )(fx_t, kp_t)

    return (jnp.sum(tc_partial) + jnp.sum(sc_partial)) / (B * N)


# speedup vs baseline: 1.1287x; 1.0271x over previous
import functools

import jax
import jax.numpy as jnp
from jax import lax
from jax.experimental import pallas as pl
from jax.experimental.pallas import tpu as pltpu
from jax.experimental.pallas import tpu_sc as plsc

K_NN = 16
B = 4
N = 2048
TC_SHARE = 1120
ROWS = 280

L = 16
NW = 32
W_PER_BATCH = NW // B
SC_SHARE = N - TC_SHARE
ROWS_PER_W = SC_SHARE // W_PER_BATCH
NV = N // L
CAP = 256


def _tc_block(fx_nat_ref, fx_t_ref, kp_nat_ref, kp_t_ref, out_ref):
    i = pl.program_id(1)
    r0 = i * ROWS

    fxb = fx_t_ref[0]
    kpb = kp_t_ref[0]
    fx_rows = fx_nat_ref[0, pl.ds(r0, ROWS), :]
    kp_rows = kp_nat_ref[0, pl.ds(r0, ROWS), :]

    def sq_dist(rows, pts_t):
        a2 = jnp.sum(rows * rows, axis=1, keepdims=True)
        b2 = jnp.sum(pts_t * pts_t, axis=0, keepdims=True)
        ab = jax.lax.dot_general(
            rows, pts_t, (((1,), (0,)), ((), ())),
            preferred_element_type=jnp.float32)
        return jnp.maximum(a2 + b2 - 2.0 * ab, 0.0)

    d_fixed = sq_dist(fx_rows, fxb)
    d_kpt = sq_dist(kp_rows, kpb)

    m = jnp.min(d_fixed, axis=1, keepdims=True)
    for _ in range(K_NN - 1):
        m = jnp.min(jnp.where(d_fixed > m, d_fixed, jnp.inf),
                    axis=1, keepdims=True)

    diff = d_fixed - d_kpt
    s = diff * diff
    le = d_fixed <= m
    eqm = d_fixed == m
    s_le = jnp.sum(jnp.where(le, s, 0.0), axis=1, keepdims=True)
    s_eq = jnp.sum(jnp.where(eqm, s, 0.0), axis=1, keepdims=True)
    cnt_le = jnp.sum(jnp.where(le, 1.0, 0.0), axis=1, keepdims=True)
    cnt_eq = jnp.sum(jnp.where(eqm, 1.0, 0.0), axis=1, keepdims=True)
    cnt_lt = cnt_le - cnt_eq
    w = jnp.clip((K_NN - cnt_lt) / jnp.maximum(cnt_eq, 1.0), 0.0, 1.0)
    loss = jnp.sum(s_le - (1.0 - w) * s_eq)
    out_ref[...] = loss.reshape(1, 1, 1, 1)


def _sc_body(fx_t_h, kp_t_h, out_h,
             fxv, kpv, dbuf, candv, candi, acc):
    wid = lax.axis_index("s") * 2 + lax.axis_index("c")
    b = wid // W_PER_BATCH
    row0 = TC_SHARE + (wid % W_PER_BATCH) * ROWS_PER_W

    pltpu.sync_copy(fx_t_h.at[b], fxv)
    pltpu.sync_copy(kp_t_h.at[b], kpv)
    zero_i = jnp.zeros((L,), jnp.int32)
    one_i = jnp.full((L,), 1, jnp.int32)
    two_i = jnp.full((L,), 2, jnp.int32)

    lane = lax.iota(jnp.int32, L)
    inf16 = jnp.full((L,), jnp.inf, jnp.float32)
    zero16 = jnp.zeros((L,), jnp.float32)
    for j in range(CAP // L):
        candi[pl.ds(j * L, L)] = lax.iota(jnp.int32, L)

    acc[...] = zero16

    def row_body(r, acc_v):
        ridx = jnp.full((L,), row0 + r, jnp.int32)
        xi = plsc.load_gather(fxv, [zero_i, ridx])
        yi = plsc.load_gather(fxv, [one_i, ridx])
        zi = plsc.load_gather(fxv, [two_i, ridx])

        @plsc.parallel_loop(0, N, step=L, unroll=16, carry=inf16)
        def g(v, gc):
            dx = fxv[0, pl.ds(v, L)] - xi
            dy = fxv[1, pl.ds(v, L)] - yi
            dz = fxv[2, pl.ds(v, L)] - zi
            d = dx * dx + dy * dy + dz * dz
            dbuf[pl.ds(v, L)] = d
            return jnp.minimum(gc, d)
        t = jnp.full((L,), jnp.max(g), jnp.float32)

        for j in range(CAP // L):
            candv[pl.ds(j * L, L)] = inf16

        @plsc.parallel_loop(0, N, step=L, unroll=8,
                            carry=jnp.zeros((L,), jnp.int32))
        def off(v, offc):
            d = dbuf[pl.ds(v, L)]
            m = d <= t
            pos = offc + plsc.cumsum(jnp.ones((L,), jnp.int32), mask=m) - 1
            m = jnp.logical_and(m, pos < CAP)
            plsc.store_scatter(candv, [pos], d, mask=m)
            plsc.store_scatter(candi, [pos], v + lane, mask=m)
            return offc + plsc.all_reduce_population_count(m)
        cnt = jnp.max(off)
        nv = jnp.minimum((cnt + L - 1) // L, CAP // L)

        bk, bv = plsc.sort_key_val(candv[pl.ds(0, L)], candi[pl.ds(0, L)])

        def merge_body(j, carry):
            bk, bv = carry
            ck, cv = plsc.sort_key_val(candv[pl.ds(j * L, L)],
                                       candi[pl.ds(j * L, L)])
            ck = lax.rev(ck, (0,))
            cv = lax.rev(cv, (0,))
            m = bk <= ck
            lo_k = jnp.where(m, bk, ck)
            lo_v = jnp.where(m, bv, cv)
            r = plsc.sort_key_val(lo_k, lo_v)
            return (r[0], r[1])

        bk, bv = lax.fori_loop(1, nv, merge_body, (bk, bv))

        kxi = plsc.load_gather(kpv, [zero_i, ridx])
        kyi = plsc.load_gather(kpv, [one_i, ridx])
        kzi = plsc.load_gather(kpv, [two_i, ridx])
        gx = plsc.load_gather(kpv, [zero_i, bv])
        gy = plsc.load_gather(kpv, [one_i, bv])
        gz = plsc.load_gather(kpv, [two_i, bv])
        ex = gx - kxi
        ey = gy - kyi
        ez = gz - kzi
        dk = ex * ex + ey * ey + ez * ez
        diff = bk - dk
        return acc_v + diff * diff

    acc_v = lax.fori_loop(0, ROWS_PER_W, row_body, zero16)
    acc[...] = acc_v
    pltpu.sync_copy(acc, out_h.at[wid])


def kernel(keypt, fixed_frame):
    fx_t = jnp.swapaxes(fixed_frame, 1, 2)
    kp_t = jnp.swapaxes(keypt, 1, 2)

    tc_partial = pl.pallas_call(
        _tc_block,
        grid=(B, TC_SHARE // ROWS),
        in_specs=[
            pl.BlockSpec((1, N, 3), lambda b, i: (b, 0, 0)),
            pl.BlockSpec((1, 3, N), lambda b, i: (b, 0, 0)),
            pl.BlockSpec((1, N, 3), lambda b, i: (b, 0, 0)),
            pl.BlockSpec((1, 3, N), lambda b, i: (b, 0, 0)),
        ],
        out_specs=pl.BlockSpec((1, 1, 1, 1), lambda b, i: (b, i, 0, 0)),
        out_shape=jax.ShapeDtypeStruct((B, TC_SHARE // ROWS, 1, 1),
                                       jnp.float32),
    )(fixed_frame, fx_t, keypt, kp_t)

    mesh = plsc.VectorSubcoreMesh(core_axis_name="c", subcore_axis_name="s")
    sc_partial = pl.kernel(
        _sc_body,
        mesh=mesh,
        out_type=jax.ShapeDtypeStruct((NW, L), jnp.float32),
        scratch_types=[
            pltpu.VMEM((3, N), jnp.float32),
            pltpu.VMEM((3, N), jnp.float32),
            pltpu.VMEM((N,), jnp.float32),
            pltpu.VMEM((CAP,), jnp.float32),
            pltpu.VMEM((CAP,), jnp.int32),
            pltpu.VMEM((L,), jnp.float32),
        ],
        compiler_params=pltpu.CompilerParams(needs_layout_passes=False),
    )(fx_t, kp_t)

    return (jnp.sum(tc_partial) + jnp.sum(sc_partial)) / (B * N)


# speedup vs baseline: 1.1592x; 1.0270x over previous
import functools

import jax
import jax.numpy as jnp
from jax import lax
from jax.experimental import pallas as pl
from jax.experimental.pallas import tpu as pltpu
from jax.experimental.pallas import tpu_sc as plsc

K_NN = 16
B = 4
N = 2048
TC_SHARE = 1152
ROWS = 288

L = 16
NW = 32
W_PER_BATCH = NW // B
SC_SHARE = N - TC_SHARE
ROWS_PER_W = SC_SHARE // W_PER_BATCH
NV = N // L
CAP = 256


def _tc_block(fx_nat_ref, fx_t_ref, kp_nat_ref, kp_t_ref, out_ref):
    i = pl.program_id(1)
    r0 = i * ROWS

    fxb = fx_t_ref[0]
    kpb = kp_t_ref[0]
    fx_rows = fx_nat_ref[0, pl.ds(r0, ROWS), :]
    kp_rows = kp_nat_ref[0, pl.ds(r0, ROWS), :]

    def sq_dist(rows, pts_t):
        a2 = jnp.sum(rows * rows, axis=1, keepdims=True)
        b2 = jnp.sum(pts_t * pts_t, axis=0, keepdims=True)
        ab = jax.lax.dot_general(
            rows, pts_t, (((1,), (0,)), ((), ())),
            preferred_element_type=jnp.float32)
        return jnp.maximum(a2 + b2 - 2.0 * ab, 0.0)

    d_fixed = sq_dist(fx_rows, fxb)
    d_kpt = sq_dist(kp_rows, kpb)

    m = jnp.min(d_fixed, axis=1, keepdims=True)
    for _ in range(K_NN - 1):
        m = jnp.min(jnp.where(d_fixed > m, d_fixed, jnp.inf),
                    axis=1, keepdims=True)

    diff = d_fixed - d_kpt
    s = diff * diff
    le = d_fixed <= m
    eqm = d_fixed == m
    s_le = jnp.sum(jnp.where(le, s, 0.0), axis=1, keepdims=True)
    s_eq = jnp.sum(jnp.where(eqm, s, 0.0), axis=1, keepdims=True)
    cnt_le = jnp.sum(jnp.where(le, 1.0, 0.0), axis=1, keepdims=True)
    cnt_eq = jnp.sum(jnp.where(eqm, 1.0, 0.0), axis=1, keepdims=True)
    cnt_lt = cnt_le - cnt_eq
    w = jnp.clip((K_NN - cnt_lt) / jnp.maximum(cnt_eq, 1.0), 0.0, 1.0)
    loss = jnp.sum(s_le - (1.0 - w) * s_eq)
    out_ref[...] = loss.reshape(1, 1, 1, 1)


def _sc_body(fx_t_h, kp_t_h, out_h,
             fxv, kpv, dbuf, candv, candi, acc):
    wid = lax.axis_index("s") * 2 + lax.axis_index("c")
    b = wid // W_PER_BATCH
    row0 = TC_SHARE + (wid % W_PER_BATCH) * ROWS_PER_W

    pltpu.sync_copy(fx_t_h.at[b], fxv)
    pltpu.sync_copy(kp_t_h.at[b], kpv)
    zero_i = jnp.zeros((L,), jnp.int32)
    one_i = jnp.full((L,), 1, jnp.int32)
    two_i = jnp.full((L,), 2, jnp.int32)

    lane = lax.iota(jnp.int32, L)
    inf16 = jnp.full((L,), jnp.inf, jnp.float32)
    zero16 = jnp.zeros((L,), jnp.float32)
    for j in range(CAP // L):
        candi[pl.ds(j * L, L)] = lax.iota(jnp.int32, L)

    acc[...] = zero16

    def row_body(r, acc_v):
        ridx = jnp.full((L,), row0 + r, jnp.int32)
        xi = plsc.load_gather(fxv, [zero_i, ridx])
        yi = plsc.load_gather(fxv, [one_i, ridx])
        zi = plsc.load_gather(fxv, [two_i, ridx])

        @plsc.parallel_loop(0, N, step=L, unroll=16, carry=inf16)
        def g(v, gc):
            dx = fxv[0, pl.ds(v, L)] - xi
            dy = fxv[1, pl.ds(v, L)] - yi
            dz = fxv[2, pl.ds(v, L)] - zi
            d = dx * dx + dy * dy + dz * dz
            dbuf[pl.ds(v, L)] = d
            return jnp.minimum(gc, d)
        t = jnp.full((L,), jnp.max(g), jnp.float32)

        for j in range(CAP // L):
            candv[pl.ds(j * L, L)] = inf16

        @plsc.parallel_loop(0, N, step=L, unroll=8,
                            carry=jnp.zeros((L,), jnp.int32))
        def off(v, offc):
            d = dbuf[pl.ds(v, L)]
            m = d <= t
            pos = offc + plsc.cumsum(jnp.ones((L,), jnp.int32), mask=m) - 1
            m = jnp.logical_and(m, pos < CAP)
            plsc.store_scatter(candv, [pos], d, mask=m)
            plsc.store_scatter(candi, [pos], v + lane, mask=m)
            return offc + plsc.all_reduce_population_count(m)
        cnt = jnp.max(off)
        nv = jnp.minimum((cnt + L - 1) // L, CAP // L)

        bk, bv = plsc.sort_key_val(candv[pl.ds(0, L)], candi[pl.ds(0, L)])

        def merge_body(j, carry):
            bk, bv = carry
            ck, cv = plsc.sort_key_val(candv[pl.ds(j * L, L)],
                                       candi[pl.ds(j * L, L)])
            ck = lax.rev(ck, (0,))
            cv = lax.rev(cv, (0,))
            m = bk <= ck
            lo_k = jnp.where(m, bk, ck)
            lo_v = jnp.where(m, bv, cv)
            r = plsc.sort_key_val(lo_k, lo_v)
            return (r[0], r[1])

        bk, bv = lax.fori_loop(1, nv, merge_body, (bk, bv))

        kxi = plsc.load_gather(kpv, [zero_i, ridx])
        kyi = plsc.load_gather(kpv, [one_i, ridx])
        kzi = plsc.load_gather(kpv, [two_i, ridx])
        gx = plsc.load_gather(kpv, [zero_i, bv])
        gy = plsc.load_gather(kpv, [one_i, bv])
        gz = plsc.load_gather(kpv, [two_i, bv])
        ex = gx - kxi
        ey = gy - kyi
        ez = gz - kzi
        dk = ex * ex + ey * ey + ez * ez
        diff = bk - dk
        return acc_v + diff * diff

    acc_v = lax.fori_loop(0, ROWS_PER_W, row_body, zero16)
    acc[...] = acc_v
    pltpu.sync_copy(acc, out_h.at[wid])


def kernel(keypt, fixed_frame):
    fx_t = jnp.swapaxes(fixed_frame, 1, 2)
    kp_t = jnp.swapaxes(keypt, 1, 2)

    tc_partial = pl.pallas_call(
        _tc_block,
        grid=(B, TC_SHARE // ROWS),
        in_specs=[
            pl.BlockSpec((1, N, 3), lambda b, i: (b, 0, 0)),
            pl.BlockSpec((1, 3, N), lambda b, i: (b, 0, 0)),
            pl.BlockSpec((1, N, 3), lambda b, i: (b, 0, 0)),
            pl.BlockSpec((1, 3, N), lambda b, i: (b, 0, 0)),
        ],
        out_specs=pl.BlockSpec((1, 1, 1, 1), lambda b, i: (b, i, 0, 0)),
        out_shape=jax.ShapeDtypeStruct((B, TC_SHARE // ROWS, 1, 1),
                                       jnp.float32),
    )(fixed_frame, fx_t, keypt, kp_t)

    mesh = plsc.VectorSubcoreMesh(core_axis_name="c", subcore_axis_name="s")
    sc_partial = pl.kernel(
        _sc_body,
        mesh=mesh,
        out_type=jax.ShapeDtypeStruct((NW, L), jnp.float32),
        scratch_types=[
            pltpu.VMEM((3, N), jnp.float32),
            pltpu.VMEM((3, N), jnp.float32),
            pltpu.VMEM((N,), jnp.float32),
            pltpu.VMEM((CAP,), jnp.float32),
            pltpu.VMEM((CAP,), jnp.int32),
            pltpu.VMEM((L,), jnp.float32),
        ],
        compiler_params=pltpu.CompilerParams(needs_layout_passes=False),
    )(fx_t, kp_t)

    return (jnp.sum(tc_partial) + jnp.sum(sc_partial)) / (B * N)


# speedup vs baseline: 1.1595x; 1.0002x over previous
import functools

import jax
import jax.numpy as jnp
from jax import lax
from jax.experimental import pallas as pl
from jax.experimental.pallas import tpu as pltpu
from jax.experimental.pallas import tpu_sc as plsc

K_NN = 16
B = 4
N = 2048
TC_SHARE = 1216
ROWS = 304

L = 16
NW = 32
W_PER_BATCH = NW // B
SC_SHARE = N - TC_SHARE
ROWS_PER_W = SC_SHARE // W_PER_BATCH
NV = N // L
CAP = 256


def _tc_block(fx_nat_ref, fx_t_ref, kp_nat_ref, kp_t_ref, out_ref):
    i = pl.program_id(1)
    r0 = i * ROWS

    fxb = fx_t_ref[0]
    kpb = kp_t_ref[0]
    fx_rows = fx_nat_ref[0, pl.ds(r0, ROWS), :]
    kp_rows = kp_nat_ref[0, pl.ds(r0, ROWS), :]

    def sq_dist(rows, pts_t):
        a2 = jnp.sum(rows * rows, axis=1, keepdims=True)
        b2 = jnp.sum(pts_t * pts_t, axis=0, keepdims=True)
        ab = jax.lax.dot_general(
            rows, pts_t, (((1,), (0,)), ((), ())),
            preferred_element_type=jnp.float32)
        return jnp.maximum(a2 + b2 - 2.0 * ab, 0.0)

    d_fixed = sq_dist(fx_rows, fxb)
    d_kpt = sq_dist(kp_rows, kpb)

    m = jnp.min(d_fixed, axis=1, keepdims=True)
    for _ in range(K_NN - 1):
        m = jnp.min(jnp.where(d_fixed > m, d_fixed, jnp.inf),
                    axis=1, keepdims=True)

    diff = d_fixed - d_kpt
    s = diff * diff
    le = d_fixed <= m
    eqm = d_fixed == m
    s_le = jnp.sum(jnp.where(le, s, 0.0), axis=1, keepdims=True)
    s_eq = jnp.sum(jnp.where(eqm, s, 0.0), axis=1, keepdims=True)
    cnt_le = jnp.sum(jnp.where(le, 1.0, 0.0), axis=1, keepdims=True)
    cnt_eq = jnp.sum(jnp.where(eqm, 1.0, 0.0), axis=1, keepdims=True)
    cnt_lt = cnt_le - cnt_eq
    w = jnp.clip((K_NN - cnt_lt) / jnp.maximum(cnt_eq, 1.0), 0.0, 1.0)
    loss = jnp.sum(s_le - (1.0 - w) * s_eq)
    out_ref[...] = loss.reshape(1, 1, 1, 1)


def _sc_body(fx_t_h, kp_t_h, out_h,
             fxv, kpv, dbuf, candv, candi, acc):
    wid = lax.axis_index("s") * 2 + lax.axis_index("c")
    b = wid // W_PER_BATCH
    row0 = TC_SHARE + (wid % W_PER_BATCH) * ROWS_PER_W

    pltpu.sync_copy(fx_t_h.at[b], fxv)
    pltpu.sync_copy(kp_t_h.at[b], kpv)
    zero_i = jnp.zeros((L,), jnp.int32)
    one_i = jnp.full((L,), 1, jnp.int32)
    two_i = jnp.full((L,), 2, jnp.int32)

    lane = lax.iota(jnp.int32, L)
    inf16 = jnp.full((L,), jnp.inf, jnp.float32)
    zero16 = jnp.zeros((L,), jnp.float32)
    for j in range(CAP // L):
        candi[pl.ds(j * L, L)] = lax.iota(jnp.int32, L)

    acc[...] = zero16

    def row_body(r, acc_v):
        ridx = jnp.full((L,), row0 + r, jnp.int32)
        xi = plsc.load_gather(fxv, [zero_i, ridx])
        yi = plsc.load_gather(fxv, [one_i, ridx])
        zi = plsc.load_gather(fxv, [two_i, ridx])

        @plsc.parallel_loop(0, N, step=L, unroll=16, carry=inf16)
        def g(v, gc):
            dx = fxv[0, pl.ds(v, L)] - xi
            dy = fxv[1, pl.ds(v, L)] - yi
            dz = fxv[2, pl.ds(v, L)] - zi
            d = dx * dx + dy * dy + dz * dz
            dbuf[pl.ds(v, L)] = d
            return jnp.minimum(gc, d)
        t = jnp.full((L,), jnp.max(g), jnp.float32)

        for j in range(CAP // L):
            candv[pl.ds(j * L, L)] = inf16

        @plsc.parallel_loop(0, N, step=L, unroll=8,
                            carry=jnp.zeros((L,), jnp.int32))
        def off(v, offc):
            d = dbuf[pl.ds(v, L)]
            m = d <= t
            pos = offc + plsc.cumsum(jnp.ones((L,), jnp.int32), mask=m) - 1
            m = jnp.logical_and(m, pos < CAP)
            plsc.store_scatter(candv, [pos], d, mask=m)
            plsc.store_scatter(candi, [pos], v + lane, mask=m)
            return offc + plsc.all_reduce_population_count(m)
        cnt = jnp.max(off)
        nv = jnp.minimum((cnt + L - 1) // L, CAP // L)

        bk, bv = plsc.sort_key_val(candv[pl.ds(0, L)], candi[pl.ds(0, L)])

        def merge_body(j, carry):
            bk, bv = carry
            ck, cv = plsc.sort_key_val(candv[pl.ds(j * L, L)],
                                       candi[pl.ds(j * L, L)])
            ck = lax.rev(ck, (0,))
            cv = lax.rev(cv, (0,))
            m = bk <= ck
            lo_k = jnp.where(m, bk, ck)
            lo_v = jnp.where(m, bv, cv)
            r = plsc.sort_key_val(lo_k, lo_v)
            return (r[0], r[1])

        bk, bv = lax.fori_loop(1, nv, merge_body, (bk, bv))

        kxi = plsc.load_gather(kpv, [zero_i, ridx])
        kyi = plsc.load_gather(kpv, [one_i, ridx])
        kzi = plsc.load_gather(kpv, [two_i, ridx])
        gx = plsc.load_gather(kpv, [zero_i, bv])
        gy = plsc.load_gather(kpv, [one_i, bv])
        gz = plsc.load_gather(kpv, [two_i, bv])
        ex = gx - kxi
        ey = gy - kyi
        ez = gz - kzi
        dk = ex * ex + ey * ey + ez * ez
        diff = bk - dk
        return acc_v + diff * diff

    acc_v = lax.fori_loop(0, ROWS_PER_W, row_body, zero16)
    acc[...] = acc_v
    pltpu.sync_copy(acc, out_h.at[wid])


def kernel(keypt, fixed_frame):
    fx_t = jnp.swapaxes(fixed_frame, 1, 2)
    kp_t = jnp.swapaxes(keypt, 1, 2)

    tc_partial = pl.pallas_call(
        _tc_block,
        grid=(B, TC_SHARE // ROWS),
        in_specs=[
            pl.BlockSpec((1, N, 3), lambda b, i: (b, 0, 0)),
            pl.BlockSpec((1, 3, N), lambda b, i: (b, 0, 0)),
            pl.BlockSpec((1, N, 3), lambda b, i: (b, 0, 0)),
            pl.BlockSpec((1, 3, N), lambda b, i: (b, 0, 0)),
        ],
        out_specs=pl.BlockSpec((1, 1, 1, 1), lambda b, i: (b, i, 0, 0)),
        out_shape=jax.ShapeDtypeStruct((B, TC_SHARE // ROWS, 1, 1),
                                       jnp.float32),
    )(fixed_frame, fx_t, keypt, kp_t)

    mesh = plsc.VectorSubcoreMesh(core_axis_name="c", subcore_axis_name="s")
    sc_partial = pl.kernel(
        _sc_body,
        mesh=mesh,
        out_type=jax.ShapeDtypeStruct((NW, L), jnp.float32),
        scratch_types=[
            pltpu.VMEM((3, N), jnp.float32),
            pltpu.VMEM((3, N), jnp.float32),
            pltpu.VMEM((N,), jnp.float32),
            pltpu.VMEM((CAP,), jnp.float32),
            pltpu.VMEM((CAP,), jnp.int32),
            pltpu.VMEM((L,), jnp.float32),
        ],
        compiler_params=pltpu.CompilerParams(needs_layout_passes=False),
    )(fx_t, kp_t)

    return (jnp.sum(tc_partial) + jnp.sum(sc_partial)) / (B * N)


# speedup vs baseline: 1.1597x; 1.0002x over previous
import jax
import jax.numpy as jnp
from jax import lax
from jax.experimental import pallas as pl
from jax.experimental.pallas import tpu as pltpu
from jax.experimental.pallas import tpu_sc as plsc

K_NN = 16
B = 4
N = 2048
TC_SHARE = 1216
ROWS = 304

L = 16
NW = 32
W_PER_BATCH = NW // B
SC_SHARE = N - TC_SHARE
ROWS_PER_W = SC_SHARE // W_PER_BATCH
NV = N // L
CAP = 256


def _tc_block(fx_nat_ref, fx_t_ref, kp_nat_ref, kp_t_ref, out_ref):
    i = pl.program_id(1)
    r0 = i * ROWS

    fxb = fx_t_ref[0]
    kpb = kp_t_ref[0]
    fx_rows = fx_nat_ref[0, pl.ds(r0, ROWS), :]
    kp_rows = kp_nat_ref[0, pl.ds(r0, ROWS), :]

    def sq_dist(rows, pts_t):
        a2 = jnp.sum(rows * rows, axis=1, keepdims=True)
        b2 = jnp.sum(pts_t * pts_t, axis=0, keepdims=True)
        ab = jax.lax.dot_general(
            rows, pts_t, (((1,), (0,)), ((), ())),
            preferred_element_type=jnp.float32)
        return jnp.maximum(a2 + b2 - 2.0 * ab, 0.0)

    d_fixed = sq_dist(fx_rows, fxb)
    d_kpt = sq_dist(kp_rows, kpb)

    m = jnp.min(d_fixed, axis=1, keepdims=True)
    for _ in range(K_NN - 1):
        m = jnp.min(jnp.where(d_fixed > m, d_fixed, jnp.inf),
                    axis=1, keepdims=True)

    diff = d_fixed - d_kpt
    s = diff * diff
    le = d_fixed <= m
    eqm = d_fixed == m
    s_le = jnp.sum(jnp.where(le, s, 0.0), axis=1, keepdims=True)
    s_eq = jnp.sum(jnp.where(eqm, s, 0.0), axis=1, keepdims=True)
    cnt_le = jnp.sum(jnp.where(le, 1.0, 0.0), axis=1, keepdims=True)
    cnt_eq = jnp.sum(jnp.where(eqm, 1.0, 0.0), axis=1, keepdims=True)
    cnt_lt = cnt_le - cnt_eq
    w = jnp.clip((K_NN - cnt_lt) / jnp.maximum(cnt_eq, 1.0), 0.0, 1.0)
    loss = jnp.sum(s_le - (1.0 - w) * s_eq)
    out_ref[...] = loss.reshape(1, 1, 1, 1)


def _sc_body(fx_t_h, kp_t_h, out_h,
             fxv, kpv, dbuf, candv, candi, acc):
    wid = lax.axis_index("s") * 2 + lax.axis_index("c")
    b = wid // W_PER_BATCH
    row0 = TC_SHARE + (wid % W_PER_BATCH) * ROWS_PER_W

    pltpu.sync_copy(fx_t_h.at[b], fxv)
    pltpu.sync_copy(kp_t_h.at[b], kpv)
    zero_i = jnp.zeros((L,), jnp.int32)
    one_i = jnp.full((L,), 1, jnp.int32)
    two_i = jnp.full((L,), 2, jnp.int32)

    lane = lax.iota(jnp.int32, L)
    inf16 = jnp.full((L,), jnp.inf, jnp.float32)
    zero16 = jnp.zeros((L,), jnp.float32)
    for j in range(CAP // L):
        candi[pl.ds(j * L, L)] = lax.iota(jnp.int32, L)

    acc[...] = zero16

    def row_body(r, acc_v):
        ridx = jnp.full((L,), row0 + r, jnp.int32)
        xi = plsc.load_gather(fxv, [zero_i, ridx])
        yi = plsc.load_gather(fxv, [one_i, ridx])
        zi = plsc.load_gather(fxv, [two_i, ridx])

        @plsc.parallel_loop(0, N, step=L, unroll=16, carry=inf16)
        def g(v, gc):
            dx = fxv[0, pl.ds(v, L)] - xi
            dy = fxv[1, pl.ds(v, L)] - yi
            dz = fxv[2, pl.ds(v, L)] - zi
            d = dx * dx + dy * dy + dz * dz
            dbuf[pl.ds(v, L)] = d
            return jnp.minimum(gc, d)
        t = jnp.full((L,), jnp.max(g), jnp.float32)

        for j in range(CAP // L):
            candv[pl.ds(j * L, L)] = inf16

        @plsc.parallel_loop(0, N, step=L, unroll=8,
                            carry=jnp.zeros((L,), jnp.int32))
        def off(v, offc):
            d = dbuf[pl.ds(v, L)]
            m = d <= t
            pos = offc + plsc.cumsum(jnp.ones((L,), jnp.int32), mask=m) - 1
            m = jnp.logical_and(m, pos < CAP)
            plsc.store_scatter(candv, [pos], d, mask=m)
            plsc.store_scatter(candi, [pos], v + lane, mask=m)
            return offc + plsc.all_reduce_population_count(m)
        cnt = jnp.max(off)
        nv = jnp.minimum((cnt + L - 1) // L, CAP // L)

        bk, bv = plsc.sort_key_val(candv[pl.ds(0, L)], candi[pl.ds(0, L)])

        def merge_body(j, carry):
            bk, bv = carry
            ck, cv = plsc.sort_key_val(candv[pl.ds(j * L, L)],
                                       candi[pl.ds(j * L, L)])
            ck = lax.rev(ck, (0,))
            cv = lax.rev(cv, (0,))
            m = bk <= ck
            lo_k = jnp.where(m, bk, ck)
            lo_v = jnp.where(m, bv, cv)
            r = plsc.sort_key_val(lo_k, lo_v)
            return (r[0], r[1])

        bk, bv = lax.fori_loop(1, nv, merge_body, (bk, bv))

        kxi = plsc.load_gather(kpv, [zero_i, ridx])
        kyi = plsc.load_gather(kpv, [one_i, ridx])
        kzi = plsc.load_gather(kpv, [two_i, ridx])
        gx = plsc.load_gather(kpv, [zero_i, bv])
        gy = plsc.load_gather(kpv, [one_i, bv])
        gz = plsc.load_gather(kpv, [two_i, bv])
        ex = gx - kxi
        ey = gy - kyi
        ez = gz - kzi
        dk = ex * ex + ey * ey + ez * ez
        diff = bk - dk
        return acc_v + diff * diff

    acc_v = lax.fori_loop(0, ROWS_PER_W, row_body, zero16)
    acc[...] = acc_v
    pltpu.sync_copy(acc, out_h.at[wid])


def kernel(keypt, fixed_frame):
    fx_t = jnp.swapaxes(fixed_frame, 1, 2)
    kp_t = jnp.swapaxes(keypt, 1, 2)

    tc_partial = pl.pallas_call(
        _tc_block,
        grid=(B, TC_SHARE // ROWS),
        in_specs=[
            pl.BlockSpec((1, N, 3), lambda b, i: (b, 0, 0)),
            pl.BlockSpec((1, 3, N), lambda b, i: (b, 0, 0)),
            pl.BlockSpec((1, N, 3), lambda b, i: (b, 0, 0)),
            pl.BlockSpec((1, 3, N), lambda b, i: (b, 0, 0)),
        ],
        out_specs=pl.BlockSpec((1, 1, 1, 1), lambda b, i: (b, i, 0, 0)),
        out_shape=jax.ShapeDtypeStruct((B, TC_SHARE // ROWS, 1, 1),
                                       jnp.float32),
    )(fixed_frame, fx_t, keypt, kp_t)

    mesh = plsc.VectorSubcoreMesh(core_axis_name="c", subcore_axis_name="s")
    sc_partial = pl.kernel(
        _sc_body,
        mesh=mesh,
        out_type=jax.ShapeDtypeStruct((NW, L), jnp.float32),
        scratch_types=[
            pltpu.VMEM((3, N), jnp.float32),
            pltpu.VMEM((3, N), jnp.float32),
            pltpu.VMEM((N,), jnp.float32),
            pltpu.VMEM((CAP,), jnp.float32),
            pltpu.VMEM((CAP,), jnp.int32),
            pltpu.VMEM((L,), jnp.float32),
        ],
        compiler_params=pltpu.CompilerParams(needs_layout_passes=False),
    )(fx_t, kp_t)

    return (jnp.sum(tc_partial) + jnp.sum(sc_partial)) / (B * N)
